# Initial kernel scaffold; baseline (speedup 1.0000x reference)
import jax
import jax.numpy as jnp
from jax.experimental import pallas as pl


def kernel(x, edge_weight, params, edge_index):
    raise NotImplementedError("write your pallas kernel here")


# speedup vs baseline: 20.6730x; 20.6730x over previous
import jax
import jax.numpy as jnp
from jax import lax
from jax.experimental import pallas as pl
from jax.experimental.pallas import tpu as pltpu
from jax.experimental.pallas import tpu_sc as plsc

N = 10000
E = 160000
NEG = 0.2
F32 = jnp.float32
I32 = jnp.int32

NP = 10240
NC, NS = 2, 16
NW = NC * NS
CH = 128
EW = 5120
NJ = EW // CH
EP = NW * EW
RT = NP // NS
BLK = 512
GRID = NP // BLK

_MESH = plsc.VectorSubcoreMesh(
    core_axis_name="c", subcore_axis_name="s", num_cores=NC, num_subcores=NS)

_SC_PARAMS = pltpu.CompilerParams(needs_layout_passes=False)

_HIGH = lax.Precision.HIGHEST


def _leaky(v):
  return jnp.where(v >= 0, v, v * NEG)


def _dot(a, b):
  return lax.dot_general(a, b, (((1,), (0,)), ((), ())),
                         precision=_HIGH, preferred_element_type=F32)


def _sc_layer(src3, dst3, ta1, tb1, mvec, xlt, zrows, znp, d, nscale):
  def body(src_h, dst_h, ta_h, tb_h, mv_h, xl_h, z_h, zn_h,
           den_out, agg_out,
           srcv, dstv, tav, tbv, mv, exb, rows, den_sh, acc, sem):
    cid = lax.axis_index("c")
    sid = lax.axis_index("s")
    w = sid * NC + cid
    pltpu.sync_copy(src_h.at[w], srcv)
    pltpu.sync_copy(dst_h.at[w], dstv)
    pltpu.sync_copy(ta_h, tav)
    pltpu.sync_copy(tb_h, tbv)
    pltpu.sync_copy(mv_h.at[pl.ds(0, 16)], mv)
    pltpu.sync_copy(zn_h.at[pl.ds(sid * RT, RT)],
                    den_sh.at[pl.ds(sid * RT, RT)])
    for t in range(RT // 64):
      pltpu.sync_copy(z_h, acc.at[pl.ds(sid * RT + t * 64, 64)])
    plsc.subcore_barrier()

    def jbody(j, carry):
      for k in range(CH // 16):
        sv = srcv[j, pl.ds(k * 16, 16)]
        dv = dstv[j, pl.ds(k * 16, 16)]
        a = plsc.load_gather(tav, [sv])
        b = plsc.load_gather(tbv, [dv])
        m = mv[...]
        exb[pl.ds(k * 16, 16)] = jnp.exp(_leaky(a + b) - _leaky(m + b))
      pltpu.sync_copy(exb, den_sh.at[dstv.at[j]], add=True)
      pltpu.async_copy(xl_h.at[srcv.at[j]], rows, sem).wait()

      def rbody(r, c2):
        cvec = plsc.load_gather(exb, [jnp.broadcast_to(r, (16,))])
        for k2 in range(nscale):
          rows[r, pl.ds(k2 * 16, 16)] = rows[r, pl.ds(k2 * 16, 16)] * cvec
        return c2

      lax.fori_loop(0, CH, rbody, 0)
      pltpu.sync_copy(rows, acc.at[dstv.at[j]], add=True)
      return carry

    lax.fori_loop(0, NJ, jbody, 0)
    plsc.subcore_barrier()

    @pl.when(sid == 0)
    def _():
      pltpu.sync_copy(den_sh, den_out.at[cid])

    pltpu.sync_copy(acc.at[pl.ds(sid * RT, RT)],
                    agg_out.at[cid, pl.ds(sid * RT, RT)])

  return pl.kernel(
      body,
      out_type=[jax.ShapeDtypeStruct((NC, NP), F32),
                jax.ShapeDtypeStruct((NC, NP, d), F32)],
      mesh=_MESH,
      compiler_params=_SC_PARAMS,
      scratch_types=[
          pltpu.VMEM((NJ, CH), I32), pltpu.VMEM((NJ, CH), I32),
          pltpu.VMEM((NP,), F32), pltpu.VMEM((NP,), F32),
          pltpu.VMEM((16,), F32), pltpu.VMEM((CH,), F32),
          pltpu.VMEM((CH, d), F32),
          pltpu.VMEM_SHARED((NP,), F32),
          pltpu.VMEM_SHARED((NP, d), F32),
          pltpu.SemaphoreType.DMA,
      ],
  )(src3, dst3, ta1, tb1, mvec, xlt, zrows, znp)


def _mm(h, w):
  din, dout = w.shape

  def body(h_ref, w_ref, o_ref):
    o_ref[...] = _dot(h_ref[...], w_ref[...])

  return pl.pallas_call(
      body, grid=(GRID,),
      in_specs=[pl.BlockSpec((BLK, din), lambda i: (i, 0)),
                pl.BlockSpec((din, dout), lambda i: (0, 0))],
      out_specs=pl.BlockSpec((BLK, dout), lambda i: (i, 0)),
      out_shape=jax.ShapeDtypeStruct((NP, dout), F32))(h, w)


def _prep(xl, ats, atd):
  d = xl.shape[1]

  def body(xl_ref, as_ref, ad_ref, ta_ref, tb_ref, ds_ref, mv_ref):
    xlv = xl_ref[...]
    asrc = jnp.sum(xlv * as_ref[...], axis=1, keepdims=True)
    adst = jnp.sum(xlv * ad_ref[...], axis=1, keepdims=True)
    rowid = lax.broadcasted_iota(I32, (NP, 1), 0)
    ta = jnp.where(rowid < N, asrc, -1e30)
    m = jnp.max(ta)
    ta_ref[...] = ta
    tb_ref[...] = adst
    ds_ref[...] = jnp.exp(_leaky(asrc + adst) - _leaky(m + adst))
    mv_ref[...] = jnp.broadcast_to(m, (1, 128))

  return pl.pallas_call(
      body,
      out_shape=[jax.ShapeDtypeStruct((NP, 1), F32),
                 jax.ShapeDtypeStruct((NP, 1), F32),
                 jax.ShapeDtypeStruct((NP, 1), F32),
                 jax.ShapeDtypeStruct((1, 128), F32)])(
                     xl, ats.reshape(1, d), atd.reshape(1, d))


def _dense_mid(d0, d1, p0, p1, xl, dsf, b, gam, bet, wn):
  d = xl.shape[1]
  dout = wn.shape[1]

  def body(d0_ref, d1_ref, p0_ref, p1_ref, xl_ref, ds_ref, b_ref, g_ref,
           be_ref, w_ref, h_ref, o_ref):
    dsv = ds_ref[...]
    inv = 1.0 / (d0_ref[...] + d1_ref[...] + dsv + 1e-16)
    agg = ((p0_ref[...] + p1_ref[...]) * inv
           + xl_ref[...] * (dsv * inv) + b_ref[...])
    hb = agg * (g_ref[...] * (1.0 / jnp.sqrt(1.0 + 1e-5))) + be_ref[...]
    h = jnp.maximum(hb, 0.0)
    h_ref[...] = h
    o_ref[...] = _dot(h, w_ref[...])

  return pl.pallas_call(
      body, grid=(GRID,),
      in_specs=[pl.BlockSpec((BLK, 1), lambda i: (i, 0)),
                pl.BlockSpec((BLK, 1), lambda i: (i, 0)),
                pl.BlockSpec((BLK, d), lambda i: (i, 0)),
                pl.BlockSpec((BLK, d), lambda i: (i, 0)),
                pl.BlockSpec((BLK, d), lambda i: (i, 0)),
                pl.BlockSpec((BLK, 1), lambda i: (i, 0)),
                pl.BlockSpec((1, d), lambda i: (0, 0)),
                pl.BlockSpec((1, d), lambda i: (0, 0)),
                pl.BlockSpec((1, d), lambda i: (0, 0)),
                pl.BlockSpec((d, dout), lambda i: (0, 0))],
      out_specs=[pl.BlockSpec((BLK, d), lambda i: (i, 0)),
                 pl.BlockSpec((BLK, dout), lambda i: (i, 0))],
      out_shape=[jax.ShapeDtypeStruct((NP, d), F32),
                 jax.ShapeDtypeStruct((NP, dout), F32)])(
                     d0, d1, p0, p1, xl, dsf,
                     b.reshape(1, d), gam.reshape(1, d), bet.reshape(1, d), wn)


def _dense_last(d0, d1, p0, p1, xl, dsf, b, gam, bet, w4, w1, b1, w2, b2):
  def body(d0_ref, d1_ref, p0_ref, p1_ref, xl_ref, ds_ref, b_ref, g_ref,
           be_ref, w4_ref, w1_ref, b1_ref, w2_ref, b2_ref, t_ref):
    dsv = ds_ref[...]
    inv = 1.0 / (d0_ref[...] + d1_ref[...] + dsv + 1e-16)
    agg = ((p0_ref[...] + p1_ref[...]) * inv
           + xl_ref[...] * (dsv * inv) + b_ref[...])
    hb = agg * (g_ref[...] * (1.0 / jnp.sqrt(1.0 + 1e-5))) + be_ref[...]
    h = jnp.maximum(hb, 0.0)
    xl4 = _dot(h, w4_ref[...])
    t = _dot(h, w1_ref[...]) + b1_ref[...]
    t2 = _dot(t, w2_ref[...]) + b2_ref[...]
    t2 = t2 - jnp.max(t2, axis=1, keepdims=True)
    et = jnp.exp(t2)
    sm = et / jnp.sum(et, axis=1, keepdims=True)
    rowid = (lax.broadcasted_iota(I32, (BLK, 1), 0)
             + pl.program_id(0) * BLK)
    s = jnp.where(rowid < N, sm, 0.0)
    t_ref[...] = jnp.concatenate(
        [xl4, s, jnp.zeros((BLK, 64), F32)], axis=1)

  return pl.pallas_call(
      body, grid=(GRID,),
      in_specs=[pl.BlockSpec((BLK, 1), lambda i: (i, 0)),
                pl.BlockSpec((BLK, 1), lambda i: (i, 0)),
                pl.BlockSpec((BLK, 128), lambda i: (i, 0)),
                pl.BlockSpec((BLK, 128), lambda i: (i, 0)),
                pl.BlockSpec((BLK, 128), lambda i: (i, 0)),
                pl.BlockSpec((BLK, 1), lambda i: (i, 0)),
                pl.BlockSpec((1, 128), lambda i: (0, 0)),
                pl.BlockSpec((1, 128), lambda i: (0, 0)),
                pl.BlockSpec((1, 128), lambda i: (0, 0)),
                pl.BlockSpec((128, 32), lambda i: (0, 0)),
                pl.BlockSpec((128, 128), lambda i: (0, 0)),
                pl.BlockSpec((1, 128), lambda i: (0, 0)),
                pl.BlockSpec((128, 32), lambda i: (0, 0)),
                pl.BlockSpec((1, 32), lambda i: (0, 0))],
      out_specs=pl.BlockSpec((BLK, 128), lambda i: (i, 0)),
      out_shape=jax.ShapeDtypeStruct((NP, 128), F32))(
          d0, d1, p0, p1, xl, dsf, b.reshape(1, 128), gam.reshape(1, 128),
          bet.reshape(1, 128), w4, w1, b1.reshape(1, 128), w2,
          b2.reshape(1, 32))


def _finalize(t, d0, d1, p0, p1, dsf, b, eye):
  c = 32

  def body(t_ref, d0_ref, d1_ref, p0_ref, p1_ref, ds_ref, b_ref, eye_ref,
           z_ref, l_ref):
    tv = t_ref[...]
    xl4 = tv[:, 0:32]
    sv = tv[:, 32:64]
    pv = p0_ref[...] + p1_ref[...]
    dsv = ds_ref[...]
    inv = 1.0 / (d0_ref[...] + d1_ref[...] + dsv + 1e-16)
    zr = pv[:, 0:32] * inv + xl4 * (dsv * inv) + b_ref[...]
    rn = jnp.sqrt(jnp.sum(zr * zr, axis=1, keepdims=True))
    z_ref[...] = zr / jnp.maximum(rn, 1e-12)

    g = pv[:, 32:64]
    t1 = jnp.sum(g * sv)
    v = jnp.sum(g, axis=0, keepdims=True)
    vv = jnp.sum(v * v)
    m = E / 2.0
    spectral = -(t1 - vv / (2.0 * m)) / (2.0 * m)

    ss = lax.dot_general(sv, sv, (((0,), (0,)), ((), ())),
                         precision=_HIGH, preferred_element_type=F32)
    ssn = jnp.sqrt(jnp.sum(ss * ss))
    dif = ss / ssn - eye_ref[...] / jnp.sqrt(1.0 * c)
    ortho = jnp.sqrt(jnp.sum(dif * dif))

    csz = jnp.sum(sv, axis=0, keepdims=True)
    cluster = jnp.sqrt(jnp.sum(csz * csz)) / N * jnp.sqrt(1.0 * c) - 1.0

    l_ref[...] = jnp.broadcast_to(spectral + ortho + cluster, (1, 1))

  return pl.pallas_call(
      body,
      out_shape=[jax.ShapeDtypeStruct((NP, c), F32),
                 jax.ShapeDtypeStruct((1, 1), F32)])(
                     t, d0, d1, p0, p1, dsf, b.reshape(1, c), eye)


def kernel(x, edge_weight, params, edge_index):
  del edge_weight
  src = edge_index[0].astype(I32)
  dst = edge_index[1].astype(I32)
  pad = N + (jnp.arange(EP - E, dtype=I32) % (NP - N))
  src3 = jnp.concatenate([src, pad]).reshape(NW, NJ, CH)
  dst3 = jnp.concatenate([dst, pad]).reshape(NW, NJ, CH)

  xp = jnp.pad(x, ((0, NP - N), (0, 0)))
  znp = jnp.zeros((NP,), F32)
  zr128 = jnp.zeros((64, 128), F32)
  eye = jnp.eye(32, dtype=F32)
  mp = params['pool']

  xl = _mm(xp, params['conv0']['W'])
  for i in range(4):
    p = params['conv%d' % i]
    ta, tb, dsf, mvec = _prep(xl, p['att_src'], p['att_dst'])
    denp, outp = _sc_layer(src3, dst3, ta.reshape(NP), tb.reshape(NP),
                           mvec.reshape(128), xl, zr128, znp, 128, 8)
    bn = params['bn%d' % i]
    d0 = denp[0].reshape(NP, 1)
    d1 = denp[1].reshape(NP, 1)
    if i < 3:
      wn = params['conv%d' % (i + 1)]['W']
      _, xl = _dense_mid(d0, d1, outp[0], outp[1], xl, dsf,
                         p['b'], bn['gamma'], bn['beta'], wn)
    else:
      t = _dense_last(d0, d1, outp[0], outp[1], xl, dsf,
                      p['b'], bn['gamma'], bn['beta'],
                      params['conv4']['W'], mp['W1'], mp['b1'],
                      mp['W2'], mp['b2'])

  p4 = params['conv4']
  att_s4 = jnp.pad(p4['att_src'], (0, 96))
  att_d4 = jnp.pad(p4['att_dst'], (0, 96))
  ta, tb, dsf4, mvec = _prep(t, att_s4, att_d4)
  denp4, outp4 = _sc_layer(src3, dst3, ta.reshape(NP), tb.reshape(NP),
                           mvec.reshape(128), t, zr128, znp, 128, 2)

  z_full, loss = _finalize(t, denp4[0].reshape(NP, 1),
                           denp4[1].reshape(NP, 1), outp4[0], outp4[1],
                           dsf4, p4['b'], eye)
  s_full = t[:, 32:64]
  return (s_full[:N][None], z_full[:N], loss[0, 0])


# speedup vs baseline: 26.1679x; 1.2658x over previous
import jax
import jax.numpy as jnp
from jax import lax
from jax.experimental import pallas as pl
from jax.experimental.pallas import tpu as pltpu
from jax.experimental.pallas import tpu_sc as plsc

N = 10000
E = 160000
NEG = 0.2
F32 = jnp.float32
I32 = jnp.int32

NP = 10240
NC, NS = 2, 16
NW = NC * NS
CH = 128
EW = 5120
NJ = EW // CH
EP = NW * EW
RT = NP // NS
BLK = 512
GRID = NP // BLK

_MESH = plsc.VectorSubcoreMesh(
    core_axis_name="c", subcore_axis_name="s", num_cores=NC, num_subcores=NS)

_SC_PARAMS = pltpu.CompilerParams(needs_layout_passes=False)

_HIGH = lax.Precision.HIGHEST


def _leaky(v):
  return jnp.where(v >= 0, v, v * NEG)


def _dot(a, b):
  return lax.dot_general(a, b, (((1,), (0,)), ((), ())),
                         precision=_HIGH, preferred_element_type=F32)


def _sc_layer(src3, dst3, ta1, tb1, mvec, xlt, zrows, znp, d, nscale):
  def body(src_h, dst_h, ta_h, tb_h, mv_h, xl_h, z_h, zn_h,
           den_out, agg_out,
           srcv, dstv, av, bv, mv, exb, rowsa, rowsb,
           ta_sh, tb_sh, den_sh, acc, sema, semb):
    cid = lax.axis_index("c")
    sid = lax.axis_index("s")
    w = sid * NC + cid
    pltpu.sync_copy(src_h.at[w], srcv)
    pltpu.sync_copy(dst_h.at[w], dstv)
    pltpu.sync_copy(mv_h.at[pl.ds(0, 16)], mv)
    sl = pl.ds(sid * RT, RT)
    pltpu.sync_copy(ta_h.at[sl], ta_sh.at[sl])
    pltpu.sync_copy(tb_h.at[sl], tb_sh.at[sl])
    pltpu.sync_copy(zn_h.at[sl], den_sh.at[sl])
    for t in range(RT // 64):
      pltpu.sync_copy(z_h, acc.at[pl.ds(sid * RT + t * 64, 64)])
    pltpu.async_copy(xl_h.at[srcv.at[0]], rowsa, sema)
    plsc.subcore_barrier()

    def chunk(j, rows_cur, sem_cur, rows_nxt, sem_nxt):
      @pl.when(j + 1 < NJ)
      def _():
        pltpu.async_copy(xl_h.at[srcv.at[j + 1]], rows_nxt, sem_nxt)
      pltpu.sync_copy(ta_sh.at[srcv.at[j]], av)
      pltpu.sync_copy(tb_sh.at[dstv.at[j]], bv)
      m = mv[...]
      for k in range(CH // 16):
        a = av[pl.ds(k * 16, 16)]
        b = bv[pl.ds(k * 16, 16)]
        exb[pl.ds(k * 16, 16)] = jnp.exp(_leaky(a + b) - _leaky(m + b))
      pltpu.sync_copy(exb, den_sh.at[dstv.at[j]], add=True)
      pltpu.make_async_copy(xl_h.at[srcv.at[j]], rows_cur, sem_cur).wait()

      def rbody(r, c2):
        for rr in range(4):
          ri = r * 4 + rr
          cvec = plsc.load_gather(exb, [jnp.broadcast_to(ri, (16,))])
          for k2 in range(nscale):
            rows_cur[ri, pl.ds(k2 * 16, 16)] = (
                rows_cur[ri, pl.ds(k2 * 16, 16)] * cvec)
        return c2

      lax.fori_loop(0, CH // 4, rbody, 0)
      pltpu.sync_copy(rows_cur, acc.at[dstv.at[j]], add=True)

    def tbody(t, carry):
      chunk(t * 2, rowsa, sema, rowsb, semb)
      chunk(t * 2 + 1, rowsb, semb, rowsa, sema)
      return carry

    lax.fori_loop(0, NJ // 2, tbody, 0)
    plsc.subcore_barrier()

    @pl.when(sid == 0)
    def _():
      pltpu.sync_copy(den_sh, den_out.at[cid])

    pltpu.sync_copy(acc.at[sl], agg_out.at[cid, sl])

  return pl.kernel(
      body,
      out_type=[jax.ShapeDtypeStruct((NC, NP), F32),
                jax.ShapeDtypeStruct((NC, NP, d), F32)],
      mesh=_MESH,
      compiler_params=_SC_PARAMS,
      scratch_types=[
          pltpu.VMEM((NJ, CH), I32), pltpu.VMEM((NJ, CH), I32),
          pltpu.VMEM((CH,), F32), pltpu.VMEM((CH,), F32),
          pltpu.VMEM((16,), F32), pltpu.VMEM((CH,), F32),
          pltpu.VMEM((CH, d), F32), pltpu.VMEM((CH, d), F32),
          pltpu.VMEM_SHARED((NP,), F32), pltpu.VMEM_SHARED((NP,), F32),
          pltpu.VMEM_SHARED((NP,), F32),
          pltpu.VMEM_SHARED((NP, d), F32),
          pltpu.SemaphoreType.DMA, pltpu.SemaphoreType.DMA,
      ],
  )(src3, dst3, ta1, tb1, mvec, xlt, zrows, znp)


def _mm(h, w):
  din, dout = w.shape

  def body(h_ref, w_ref, o_ref):
    o_ref[...] = _dot(h_ref[...], w_ref[...])

  return pl.pallas_call(
      body, grid=(GRID,),
      in_specs=[pl.BlockSpec((BLK, din), lambda i: (i, 0)),
                pl.BlockSpec((din, dout), lambda i: (0, 0))],
      out_specs=pl.BlockSpec((BLK, dout), lambda i: (i, 0)),
      out_shape=jax.ShapeDtypeStruct((NP, dout), F32))(h, w)


def _prep(xl, ats, atd):
  d = xl.shape[1]

  def body(xl_ref, as_ref, ad_ref, ta_ref, tb_ref, ds_ref, mv_ref):
    xlv = xl_ref[...]
    asrc = jnp.sum(xlv * as_ref[...], axis=1, keepdims=True)
    adst = jnp.sum(xlv * ad_ref[...], axis=1, keepdims=True)
    rowid = lax.broadcasted_iota(I32, (NP, 1), 0)
    ta = jnp.where(rowid < N, asrc, -1e30)
    m = jnp.max(ta)
    ta_ref[...] = ta
    tb_ref[...] = adst
    ds_ref[...] = jnp.exp(_leaky(asrc + adst) - _leaky(m + adst))
    mv_ref[...] = jnp.broadcast_to(m, (1, 128))

  return pl.pallas_call(
      body,
      out_shape=[jax.ShapeDtypeStruct((NP, 1), F32),
                 jax.ShapeDtypeStruct((NP, 1), F32),
                 jax.ShapeDtypeStruct((NP, 1), F32),
                 jax.ShapeDtypeStruct((1, 128), F32)])(
                     xl, ats.reshape(1, d), atd.reshape(1, d))


def _dense_mid(d0, d1, p0, p1, xl, dsf, b, gam, bet, wn):
  d = xl.shape[1]
  dout = wn.shape[1]

  def body(d0_ref, d1_ref, p0_ref, p1_ref, xl_ref, ds_ref, b_ref, g_ref,
           be_ref, w_ref, h_ref, o_ref):
    dsv = ds_ref[...]
    inv = 1.0 / (d0_ref[...] + d1_ref[...] + dsv + 1e-16)
    agg = ((p0_ref[...] + p1_ref[...]) * inv
           + xl_ref[...] * (dsv * inv) + b_ref[...])
    hb = agg * (g_ref[...] * (1.0 / jnp.sqrt(1.0 + 1e-5))) + be_ref[...]
    h = jnp.maximum(hb, 0.0)
    h_ref[...] = h
    o_ref[...] = _dot(h, w_ref[...])

  return pl.pallas_call(
      body, grid=(GRID,),
      in_specs=[pl.BlockSpec((BLK, 1), lambda i: (i, 0)),
                pl.BlockSpec((BLK, 1), lambda i: (i, 0)),
                pl.BlockSpec((BLK, d), lambda i: (i, 0)),
                pl.BlockSpec((BLK, d), lambda i: (i, 0)),
                pl.BlockSpec((BLK, d), lambda i: (i, 0)),
                pl.BlockSpec((BLK, 1), lambda i: (i, 0)),
                pl.BlockSpec((1, d), lambda i: (0, 0)),
                pl.BlockSpec((1, d), lambda i: (0, 0)),
                pl.BlockSpec((1, d), lambda i: (0, 0)),
                pl.BlockSpec((d, dout), lambda i: (0, 0))],
      out_specs=[pl.BlockSpec((BLK, d), lambda i: (i, 0)),
                 pl.BlockSpec((BLK, dout), lambda i: (i, 0))],
      out_shape=[jax.ShapeDtypeStruct((NP, d), F32),
                 jax.ShapeDtypeStruct((NP, dout), F32)])(
                     d0, d1, p0, p1, xl, dsf,
                     b.reshape(1, d), gam.reshape(1, d), bet.reshape(1, d), wn)


def _dense_last(d0, d1, p0, p1, xl, dsf, b, gam, bet, w4, w1, b1, w2, b2):
  def body(d0_ref, d1_ref, p0_ref, p1_ref, xl_ref, ds_ref, b_ref, g_ref,
           be_ref, w4_ref, w1_ref, b1_ref, w2_ref, b2_ref, t_ref):
    dsv = ds_ref[...]
    inv = 1.0 / (d0_ref[...] + d1_ref[...] + dsv + 1e-16)
    agg = ((p0_ref[...] + p1_ref[...]) * inv
           + xl_ref[...] * (dsv * inv) + b_ref[...])
    hb = agg * (g_ref[...] * (1.0 / jnp.sqrt(1.0 + 1e-5))) + be_ref[...]
    h = jnp.maximum(hb, 0.0)
    xl4 = _dot(h, w4_ref[...])
    t = _dot(h, w1_ref[...]) + b1_ref[...]
    t2 = _dot(t, w2_ref[...]) + b2_ref[...]
    t2 = t2 - jnp.max(t2, axis=1, keepdims=True)
    et = jnp.exp(t2)
    sm = et / jnp.sum(et, axis=1, keepdims=True)
    rowid = (lax.broadcasted_iota(I32, (BLK, 1), 0)
             + pl.program_id(0) * BLK)
    s = jnp.where(rowid < N, sm, 0.0)
    t_ref[...] = jnp.concatenate(
        [xl4, s, jnp.zeros((BLK, 64), F32)], axis=1)

  return pl.pallas_call(
      body, grid=(GRID,),
      in_specs=[pl.BlockSpec((BLK, 1), lambda i: (i, 0)),
                pl.BlockSpec((BLK, 1), lambda i: (i, 0)),
                pl.BlockSpec((BLK, 128), lambda i: (i, 0)),
                pl.BlockSpec((BLK, 128), lambda i: (i, 0)),
                pl.BlockSpec((BLK, 128), lambda i: (i, 0)),
                pl.BlockSpec((BLK, 1), lambda i: (i, 0)),
                pl.BlockSpec((1, 128), lambda i: (0, 0)),
                pl.BlockSpec((1, 128), lambda i: (0, 0)),
                pl.BlockSpec((1, 128), lambda i: (0, 0)),
                pl.BlockSpec((128, 32), lambda i: (0, 0)),
                pl.BlockSpec((128, 128), lambda i: (0, 0)),
                pl.BlockSpec((1, 128), lambda i: (0, 0)),
                pl.BlockSpec((128, 32), lambda i: (0, 0)),
                pl.BlockSpec((1, 32), lambda i: (0, 0))],
      out_specs=pl.BlockSpec((BLK, 128), lambda i: (i, 0)),
      out_shape=jax.ShapeDtypeStruct((NP, 128), F32))(
          d0, d1, p0, p1, xl, dsf, b.reshape(1, 128), gam.reshape(1, 128),
          bet.reshape(1, 128), w4, w1, b1.reshape(1, 128), w2,
          b2.reshape(1, 32))


def _finalize(t, d0, d1, p0, p1, dsf, b, eye):
  c = 32

  def body(t_ref, d0_ref, d1_ref, p0_ref, p1_ref, ds_ref, b_ref, eye_ref,
           z_ref, l_ref):
    tv = t_ref[...]
    xl4 = tv[:, 0:32]
    sv = tv[:, 32:64]
    pv = p0_ref[...] + p1_ref[...]
    dsv = ds_ref[...]
    inv = 1.0 / (d0_ref[...] + d1_ref[...] + dsv + 1e-16)
    zr = pv[:, 0:32] * inv + xl4 * (dsv * inv) + b_ref[...]
    rn = jnp.sqrt(jnp.sum(zr * zr, axis=1, keepdims=True))
    z_ref[...] = zr / jnp.maximum(rn, 1e-12)

    g = pv[:, 32:64]
    t1 = jnp.sum(g * sv)
    v = jnp.sum(g, axis=0, keepdims=True)
    vv = jnp.sum(v * v)
    m = E / 2.0
    spectral = -(t1 - vv / (2.0 * m)) / (2.0 * m)

    ss = lax.dot_general(sv, sv, (((0,), (0,)), ((), ())),
                         precision=_HIGH, preferred_element_type=F32)
    ssn = jnp.sqrt(jnp.sum(ss * ss))
    dif = ss / ssn - eye_ref[...] / jnp.sqrt(1.0 * c)
    ortho = jnp.sqrt(jnp.sum(dif * dif))

    csz = jnp.sum(sv, axis=0, keepdims=True)
    cluster = jnp.sqrt(jnp.sum(csz * csz)) / N * jnp.sqrt(1.0 * c) - 1.0

    l_ref[...] = jnp.broadcast_to(spectral + ortho + cluster, (1, 1))

  return pl.pallas_call(
      body,
      out_shape=[jax.ShapeDtypeStruct((NP, c), F32),
                 jax.ShapeDtypeStruct((1, 1), F32)])(
                     t, d0, d1, p0, p1, dsf, b.reshape(1, c), eye)


def kernel(x, edge_weight, params, edge_index):
  del edge_weight
  src = edge_index[0].astype(I32)
  dst = edge_index[1].astype(I32)
  pad = N + (jnp.arange(EP - E, dtype=I32) % (NP - N))
  src3 = jnp.concatenate([src, pad]).reshape(NW, NJ, CH)
  dst3 = jnp.concatenate([dst, pad]).reshape(NW, NJ, CH)

  xp = jnp.pad(x, ((0, NP - N), (0, 0)))
  znp = jnp.zeros((NP,), F32)
  zr128 = jnp.zeros((64, 128), F32)
  eye = jnp.eye(32, dtype=F32)
  mp = params['pool']

  xl = _mm(xp, params['conv0']['W'])
  for i in range(4):
    p = params['conv%d' % i]
    ta, tb, dsf, mvec = _prep(xl, p['att_src'], p['att_dst'])
    denp, outp = _sc_layer(src3, dst3, ta.reshape(NP), tb.reshape(NP),
                           mvec.reshape(128), xl, zr128, znp, 128, 8)
    bn = params['bn%d' % i]
    d0 = denp[0].reshape(NP, 1)
    d1 = denp[1].reshape(NP, 1)
    if i < 3:
      wn = params['conv%d' % (i + 1)]['W']
      _, xl = _dense_mid(d0, d1, outp[0], outp[1], xl, dsf,
                         p['b'], bn['gamma'], bn['beta'], wn)
    else:
      t = _dense_last(d0, d1, outp[0], outp[1], xl, dsf,
                      p['b'], bn['gamma'], bn['beta'],
                      params['conv4']['W'], mp['W1'], mp['b1'],
                      mp['W2'], mp['b2'])

  p4 = params['conv4']
  att_s4 = jnp.pad(p4['att_src'], (0, 96))
  att_d4 = jnp.pad(p4['att_dst'], (0, 96))
  ta, tb, dsf4, mvec = _prep(t, att_s4, att_d4)
  denp4, outp4 = _sc_layer(src3, dst3, ta.reshape(NP), tb.reshape(NP),
                           mvec.reshape(128), t, zr128, znp, 128, 2)

  z_full, loss = _finalize(t, denp4[0].reshape(NP, 1),
                           denp4[1].reshape(NP, 1), outp4[0], outp4[1],
                           dsf4, p4['b'], eye)
  s_full = t[:, 32:64]
  return (s_full[:N][None], z_full[:N], loss[0, 0])


# speedup vs baseline: 28.7478x; 1.0986x over previous
import jax
import jax.numpy as jnp
from jax import lax
from jax.experimental import pallas as pl
from jax.experimental.pallas import tpu as pltpu
from jax.experimental.pallas import tpu_sc as plsc

N = 10000
E = 160000
NEG = 0.2
F32 = jnp.float32
I32 = jnp.int32

NP = 10240
NC, NS = 2, 16
NW = NC * NS
CH = 128
EW = 5120
NJ = EW // CH
EP = NW * EW
RT = NP // NS
BLK = 512
GRID = NP // BLK

_MESH = plsc.VectorSubcoreMesh(
    core_axis_name="c", subcore_axis_name="s", num_cores=NC, num_subcores=NS)

_SC_PARAMS = pltpu.CompilerParams(needs_layout_passes=False)

_HIGH = lax.Precision.HIGHEST


def _leaky(v):
  return jnp.where(v >= 0, v, v * NEG)


def _dot(a, b):
  return lax.dot_general(a, b, (((1,), (0,)), ((), ())),
                         precision=_HIGH, preferred_element_type=F32)


def _sc_layer(src3, dst3, ta1, tb1, mvec, xlt, zrows, znp, d, nscale):
  def body(src_h, dst_h, ta_h, tb_h, mv_h, xl_h, z_h, zn_h,
           den_out, agg_out,
           srcv, dstv, av, bv, mv, exb, rowsa, rowsb,
           ta_sh, tb_sh, den_sh, acc, sema, semb, semsa, semsb):
    cid = lax.axis_index("c")
    sid = lax.axis_index("s")
    w = sid * NC + cid
    pltpu.sync_copy(src_h.at[w], srcv)
    pltpu.sync_copy(dst_h.at[w], dstv)
    pltpu.sync_copy(mv_h.at[pl.ds(0, 16)], mv)
    sl = pl.ds(sid * RT, RT)
    pltpu.sync_copy(ta_h.at[sl], ta_sh.at[sl])
    pltpu.sync_copy(tb_h.at[sl], tb_sh.at[sl])
    pltpu.sync_copy(zn_h.at[sl], den_sh.at[sl])
    for t in range(RT // 64):
      pltpu.sync_copy(z_h, acc.at[pl.ds(sid * RT + t * 64, 64)])
    pltpu.async_copy(xl_h.at[srcv.at[0]], rowsa, sema)
    plsc.subcore_barrier()

    def chunk(j, rows_cur, sem_cur, sem_s_cur, rows_nxt, sem_nxt, sem_s_nxt):
      @pl.when(j >= 1)
      def _():
        pltpu.make_async_copy(rows_nxt, acc.at[dstv.at[0]], sem_s_nxt).wait()

      @pl.when(j + 1 < NJ)
      def _():
        pltpu.async_copy(xl_h.at[srcv.at[j + 1]], rows_nxt, sem_nxt)
      pltpu.sync_copy(ta_sh.at[srcv.at[j]], av)
      pltpu.sync_copy(tb_sh.at[dstv.at[j]], bv)
      m = mv[...]
      for k in range(CH // 16):
        a = av[pl.ds(k * 16, 16)]
        b = bv[pl.ds(k * 16, 16)]
        exb[pl.ds(k * 16, 16)] = jnp.exp(_leaky(a + b) - _leaky(m + b))
      pltpu.sync_copy(exb, den_sh.at[dstv.at[j]], add=True)
      pltpu.make_async_copy(xl_h.at[srcv.at[j]], rows_cur, sem_cur).wait()

      def rbody(r, c2):
        for rr in range(4):
          ri = r * 4 + rr
          cvec = plsc.load_gather(exb, [jnp.broadcast_to(ri, (16,))])
          for k2 in range(nscale):
            rows_cur[ri, pl.ds(k2 * 16, 16)] = (
                rows_cur[ri, pl.ds(k2 * 16, 16)] * cvec)
        return c2

      lax.fori_loop(0, CH // 4, rbody, 0)
      pltpu.async_copy(rows_cur, acc.at[dstv.at[j]], sem_s_cur, add=True)

    def tbody(t, carry):
      chunk(t * 2, rowsa, sema, semsa, rowsb, semb, semsb)
      chunk(t * 2 + 1, rowsb, semb, semsb, rowsa, sema, semsa)
      return carry

    lax.fori_loop(0, NJ // 2, tbody, 0)
    pltpu.make_async_copy(rowsb, acc.at[dstv.at[0]], semsb).wait()
    plsc.subcore_barrier()

    @pl.when(sid == 0)
    def _():
      pltpu.sync_copy(den_sh, den_out.at[cid])

    pltpu.sync_copy(acc.at[sl], agg_out.at[cid, sl])

  return pl.kernel(
      body,
      out_type=[jax.ShapeDtypeStruct((NC, NP), F32),
                jax.ShapeDtypeStruct((NC, NP, d), F32)],
      mesh=_MESH,
      compiler_params=_SC_PARAMS,
      scratch_types=[
          pltpu.VMEM((NJ, CH), I32), pltpu.VMEM((NJ, CH), I32),
          pltpu.VMEM((CH,), F32), pltpu.VMEM((CH,), F32),
          pltpu.VMEM((16,), F32), pltpu.VMEM((CH,), F32),
          pltpu.VMEM((CH, d), F32), pltpu.VMEM((CH, d), F32),
          pltpu.VMEM_SHARED((NP,), F32), pltpu.VMEM_SHARED((NP,), F32),
          pltpu.VMEM_SHARED((NP,), F32),
          pltpu.VMEM_SHARED((NP, d), F32),
          pltpu.SemaphoreType.DMA, pltpu.SemaphoreType.DMA,
          pltpu.SemaphoreType.DMA, pltpu.SemaphoreType.DMA,
      ],
  )(src3, dst3, ta1, tb1, mvec, xlt, zrows, znp)


_BS = lambda r, c: pl.BlockSpec((r, c), lambda i: (i, 0))
_BC = lambda r, c: pl.BlockSpec((r, c), lambda i: (0, 0))

_TAB_OUT = [jax.ShapeDtypeStruct((NP, 1), F32),
            jax.ShapeDtypeStruct((NP, 1), F32),
            jax.ShapeDtypeStruct((1, 128), F32)]
_TAB_SPECS = [_BS(BLK, 1), _BS(BLK, 1), _BC(1, 128)]


def _emit_tabs(xl, asv, adv, ta_ref, tb_ref, mv_ref):
  asrc = jnp.sum(xl * asv, axis=1, keepdims=True)
  adst = jnp.sum(xl * adv, axis=1, keepdims=True)
  rowid = (lax.broadcasted_iota(I32, (BLK, 1), 0)
           + pl.program_id(0) * BLK)
  ta = jnp.where(rowid < N, asrc, -1e30)
  ta_ref[...] = ta
  tb_ref[...] = adst

  @pl.when(pl.program_id(0) == 0)
  def _():
    mv_ref[...] = jnp.full((1, 128), -3e38, F32)

  mv_ref[...] = jnp.maximum(mv_ref[...], jnp.max(ta))


def _self_exp(ta, tb, m):
  return jnp.exp(_leaky(ta + tb) - _leaky(m + tb))


def _dense0(x, w, ats, atd):
  def body(x_ref, w_ref, as_ref, ad_ref, xl_ref, ta_ref, tb_ref, mv_ref):
    xl = _dot(x_ref[...], w_ref[...])
    xl_ref[...] = xl
    _emit_tabs(xl, as_ref[...], ad_ref[...], ta_ref, tb_ref, mv_ref)

  return pl.pallas_call(
      body, grid=(GRID,),
      in_specs=[_BS(BLK, 128), _BC(128, 128), _BC(1, 128), _BC(1, 128)],
      out_specs=[_BS(BLK, 128)] + _TAB_SPECS,
      out_shape=[jax.ShapeDtypeStruct((NP, 128), F32)] + _TAB_OUT)(
          x, w, ats.reshape(1, 128), atd.reshape(1, 128))


def _agg_h(d0, d1, p0, p1, xl, ta, tb, m, b, g, be):
  dsf = _self_exp(ta, tb, m)
  inv = 1.0 / (d0 + d1 + dsf + 1e-16)
  agg = (p0 + p1) * inv + xl * (dsf * inv) + b
  hb = agg * (g * (1.0 / jnp.sqrt(1.0 + 1e-5))) + be
  return jnp.maximum(hb, 0.0)


def _dense_mid(d0, d1, p0, p1, xl, ta, tb, mvec, b, gam, bet, wn, atsn, atdn):
  def body(d0_ref, d1_ref, p0_ref, p1_ref, xl_ref, tai_ref, tbi_ref, mvi_ref,
           b_ref, g_ref, be_ref, w_ref, as_ref, ad_ref,
           xl2_ref, ta_ref, tb_ref, mv_ref):
    h = _agg_h(d0_ref[...], d1_ref[...], p0_ref[...], p1_ref[...],
               xl_ref[...], tai_ref[...], tbi_ref[...], mvi_ref[0, 0],
               b_ref[...], g_ref[...], be_ref[...])
    xl2 = _dot(h, w_ref[...])
    xl2_ref[...] = xl2
    _emit_tabs(xl2, as_ref[...], ad_ref[...], ta_ref, tb_ref, mv_ref)

  return pl.pallas_call(
      body, grid=(GRID,),
      in_specs=[_BS(BLK, 1), _BS(BLK, 1), _BS(BLK, 128), _BS(BLK, 128),
                _BS(BLK, 128), _BS(BLK, 1), _BS(BLK, 1), _BC(1, 128),
                _BC(1, 128), _BC(1, 128), _BC(1, 128), _BC(128, 128),
                _BC(1, 128), _BC(1, 128)],
      out_specs=[_BS(BLK, 128)] + _TAB_SPECS,
      out_shape=[jax.ShapeDtypeStruct((NP, 128), F32)] + _TAB_OUT)(
          d0, d1, p0, p1, xl, ta, tb, mvec, b.reshape(1, 128),
          gam.reshape(1, 128), bet.reshape(1, 128), wn,
          atsn.reshape(1, 128), atdn.reshape(1, 128))


def _dense_last(d0, d1, p0, p1, xl, ta, tb, mvec, b, gam, bet, w4, w1, b1,
                w2, b2, ats4, atd4):
  def body(d0_ref, d1_ref, p0_ref, p1_ref, xl_ref, tai_ref, tbi_ref, mvi_ref,
           b_ref, g_ref, be_ref, w4_ref, w1_ref, b1_ref, w2_ref, b2_ref,
           as_ref, ad_ref, t_ref, ta_ref, tb_ref, mv_ref):
    h = _agg_h(d0_ref[...], d1_ref[...], p0_ref[...], p1_ref[...],
               xl_ref[...], tai_ref[...], tbi_ref[...], mvi_ref[0, 0],
               b_ref[...], g_ref[...], be_ref[...])
    xl4 = _dot(h, w4_ref[...])
    t = _dot(h, w1_ref[...]) + b1_ref[...]
    t2 = _dot(t, w2_ref[...]) + b2_ref[...]
    t2 = t2 - jnp.max(t2, axis=1, keepdims=True)
    et = jnp.exp(t2)
    sm = et / jnp.sum(et, axis=1, keepdims=True)
    rowid = (lax.broadcasted_iota(I32, (BLK, 1), 0)
             + pl.program_id(0) * BLK)
    s = jnp.where(rowid < N, sm, 0.0)
    tv = jnp.concatenate([xl4, s, jnp.zeros((BLK, 64), F32)], axis=1)
    t_ref[...] = tv
    _emit_tabs(tv, as_ref[...], ad_ref[...], ta_ref, tb_ref, mv_ref)

  return pl.pallas_call(
      body, grid=(GRID,),
      in_specs=[_BS(BLK, 1), _BS(BLK, 1), _BS(BLK, 128), _BS(BLK, 128),
                _BS(BLK, 128), _BS(BLK, 1), _BS(BLK, 1), _BC(1, 128),
                _BC(1, 128), _BC(1, 128), _BC(1, 128), _BC(128, 32),
                _BC(128, 128), _BC(1, 128), _BC(128, 32), _BC(1, 32),
                _BC(1, 128), _BC(1, 128)],
      out_specs=[_BS(BLK, 128)] + _TAB_SPECS,
      out_shape=[jax.ShapeDtypeStruct((NP, 128), F32)] + _TAB_OUT)(
          d0, d1, p0, p1, xl, ta, tb, mvec, b.reshape(1, 128),
          gam.reshape(1, 128), bet.reshape(1, 128), w4, w1,
          b1.reshape(1, 128), w2, b2.reshape(1, 32),
          ats4.reshape(1, 128), atd4.reshape(1, 128))


def _finalize(t, d0, d1, p0, p1, ta, tb, mvec, b, eye):
  c = 32

  def body(t_ref, d0_ref, d1_ref, p0_ref, p1_ref, ta_ref, tb_ref, mv_ref,
           b_ref, eye_ref, z_ref, l_ref):
    tv = t_ref[...]
    xl4 = tv[:, 0:32]
    sv = tv[:, 32:64]
    pv = p0_ref[...] + p1_ref[...]
    dsv = _self_exp(ta_ref[...], tb_ref[...], mv_ref[0, 0])
    inv = 1.0 / (d0_ref[...] + d1_ref[...] + dsv + 1e-16)
    zr = pv[:, 0:32] * inv + xl4 * (dsv * inv) + b_ref[...]
    rn = jnp.sqrt(jnp.sum(zr * zr, axis=1, keepdims=True))
    z_ref[...] = zr / jnp.maximum(rn, 1e-12)

    g = pv[:, 32:64]
    t1 = jnp.sum(g * sv)
    v = jnp.sum(g, axis=0, keepdims=True)
    vv = jnp.sum(v * v)
    m = E / 2.0
    spectral = -(t1 - vv / (2.0 * m)) / (2.0 * m)

    ss = lax.dot_general(sv, sv, (((0,), (0,)), ((), ())),
                         precision=_HIGH, preferred_element_type=F32)
    ssn = jnp.sqrt(jnp.sum(ss * ss))
    dif = ss / ssn - eye_ref[...] / jnp.sqrt(1.0 * c)
    ortho = jnp.sqrt(jnp.sum(dif * dif))

    csz = jnp.sum(sv, axis=0, keepdims=True)
    cluster = jnp.sqrt(jnp.sum(csz * csz)) / N * jnp.sqrt(1.0 * c) - 1.0

    l_ref[...] = jnp.broadcast_to(spectral + ortho + cluster, (1, 1))

  return pl.pallas_call(
      body,
      out_shape=[jax.ShapeDtypeStruct((NP, c), F32),
                 jax.ShapeDtypeStruct((1, 1), F32)])(
                     t, d0, d1, p0, p1, ta, tb, mvec, b.reshape(1, c), eye)


def kernel(x, edge_weight, params, edge_index):
  del edge_weight
  src = edge_index[0].astype(I32)
  dst = edge_index[1].astype(I32)
  pad = N + (jnp.arange(EP - E, dtype=I32) % (NP - N))
  src3 = jnp.concatenate([src, pad]).reshape(NW, NJ, CH)
  dst3 = jnp.concatenate([dst, pad]).reshape(NW, NJ, CH)

  xp = jnp.pad(x, ((0, NP - N), (0, 0)))
  znp = jnp.zeros((NP,), F32)
  zr128 = jnp.zeros((64, 128), F32)
  eye = jnp.eye(32, dtype=F32)
  mp = params['pool']
  p4 = params['conv4']

  xl, ta, tb, mvec = _dense0(xp, params['conv0']['W'],
                             params['conv0']['att_src'],
                             params['conv0']['att_dst'])
  for i in range(4):
    p = params['conv%d' % i]
    denp, outp = _sc_layer(src3, dst3, ta.reshape(NP), tb.reshape(NP),
                           mvec.reshape(128), xl, zr128, znp, 128, 8)
    bn = params['bn%d' % i]
    d0 = denp[0].reshape(NP, 1)
    d1 = denp[1].reshape(NP, 1)
    if i < 3:
      pn = params['conv%d' % (i + 1)]
      xl, ta, tb, mvec = _dense_mid(
          d0, d1, outp[0], outp[1], xl, ta, tb, mvec, p['b'], bn['gamma'],
          bn['beta'], pn['W'], pn['att_src'], pn['att_dst'])
    else:
      t, ta, tb, mvec = _dense_last(
          d0, d1, outp[0], outp[1], xl, ta, tb, mvec, p['b'], bn['gamma'],
          bn['beta'], p4['W'], mp['W1'], mp['b1'], mp['W2'], mp['b2'],
          jnp.pad(p4['att_src'], (0, 96)), jnp.pad(p4['att_dst'], (0, 96)))

  denp4, outp4 = _sc_layer(src3, dst3, ta.reshape(NP), tb.reshape(NP),
                           mvec.reshape(128), t, zr128, znp, 128, 2)

  z_full, loss = _finalize(t, denp4[0].reshape(NP, 1),
                           denp4[1].reshape(NP, 1), outp4[0], outp4[1],
                           ta, tb, mvec, p4['b'], eye)
  s_full = t[:, 32:64]
  return (s_full[:N][None], z_full[:N], loss[0, 0])


# speedup vs baseline: 31.3337x; 1.0900x over previous
import jax
import jax.numpy as jnp
from jax import lax
from jax.experimental import pallas as pl
from jax.experimental.pallas import tpu as pltpu
from jax.experimental.pallas import tpu_sc as plsc

N = 10000
E = 160000
NEG = 0.2
F32 = jnp.float32
I32 = jnp.int32

NP = 10240
NC, NS = 2, 16
NW = NC * NS
CH = 128
EW = 5120
NJ = EW // CH
EP = NW * EW
RT = NP // NS
BLK = 512
GRID = NP // BLK

_MESH = plsc.VectorSubcoreMesh(
    core_axis_name="c", subcore_axis_name="s", num_cores=NC, num_subcores=NS)

_SC_PARAMS = pltpu.CompilerParams(needs_layout_passes=False)

_HIGH = lax.Precision.HIGHEST


def _leaky(v):
  return jnp.where(v >= 0, v, v * NEG)


def _dot(a, b):
  return lax.dot_general(a, b, (((1,), (0,)), ((), ())),
                         precision=_HIGH, preferred_element_type=F32)


def _sc_layer(src3, dst3, ta1, tb1, mvec, xlt, zrows, znp, d, nscale):
  def body(src_h, dst_h, ta_h, tb_h, mv_h, xl_h, z_h, zn_h,
           den_out, agg_out,
           srcv, dstv, ava, avb, bva, bvb, mv, exba, exbb, rowsa, rowsb,
           ta_sh, tb_sh, den_sh, acc, sema, semb, semsa, semsb,
           semta, semtb, semda, semdb):
    cid = lax.axis_index("c")
    sid = lax.axis_index("s")
    w = sid * NC + cid
    pltpu.sync_copy(src_h.at[w], srcv)
    pltpu.sync_copy(dst_h.at[w], dstv)
    pltpu.sync_copy(mv_h.at[pl.ds(0, 16)], mv)
    sl = pl.ds(sid * RT, RT)
    pltpu.sync_copy(ta_h.at[sl], ta_sh.at[sl])
    pltpu.sync_copy(tb_h.at[sl], tb_sh.at[sl])
    pltpu.sync_copy(zn_h.at[sl], den_sh.at[sl])
    for t in range(RT // 64):
      pltpu.sync_copy(z_h, acc.at[pl.ds(sid * RT + t * 64, 64)])
    pltpu.async_copy(xl_h.at[srcv.at[0]], rowsa, sema)
    plsc.subcore_barrier()
    pltpu.async_copy(ta_sh.at[srcv.at[0]], ava, semta)
    pltpu.async_copy(tb_sh.at[dstv.at[0]], bva, semta)

    def chunk(j, rows_cur, av_cur, bv_cur, exb_cur, sem_cur, sem_t_cur,
              sem_s_cur, sem_d_cur, rows_nxt, av_nxt, bv_nxt, exb_nxt,
              sem_nxt, sem_t_nxt, sem_s_nxt, sem_d_nxt):
      @pl.when(j >= 1)
      def _():
        pltpu.make_async_copy(rows_nxt, acc.at[dstv.at[0]], sem_s_nxt).wait()
        pltpu.make_async_copy(exb_nxt, den_sh.at[dstv.at[0]],
                              sem_d_nxt).wait()

      @pl.when(j + 1 < NJ)
      def _():
        pltpu.async_copy(xl_h.at[srcv.at[j + 1]], rows_nxt, sem_nxt)
        pltpu.async_copy(ta_sh.at[srcv.at[j + 1]], av_nxt, sem_t_nxt)
        pltpu.async_copy(tb_sh.at[dstv.at[j + 1]], bv_nxt, sem_t_nxt)

      pltpu.make_async_copy(ta_sh.at[srcv.at[j]], av_cur, sem_t_cur).wait()
      pltpu.make_async_copy(tb_sh.at[dstv.at[j]], bv_cur, sem_t_cur).wait()
      m = mv[...]
      for k in range(CH // 16):
        a = av_cur[pl.ds(k * 16, 16)]
        b = bv_cur[pl.ds(k * 16, 16)]
        exb_cur[pl.ds(k * 16, 16)] = jnp.exp(_leaky(a + b) - _leaky(m + b))
      pltpu.async_copy(exb_cur, den_sh.at[dstv.at[j]], sem_d_cur, add=True)
      pltpu.make_async_copy(xl_h.at[srcv.at[j]], rows_cur, sem_cur).wait()

      def rbody(r, c2):
        for rr in range(4):
          ri = r * 4 + rr
          cvec = plsc.load_gather(exb_cur, [jnp.broadcast_to(ri, (16,))])
          for k2 in range(nscale):
            rows_cur[ri, pl.ds(k2 * 16, 16)] = (
                rows_cur[ri, pl.ds(k2 * 16, 16)] * cvec)
        return c2

      lax.fori_loop(0, CH // 4, rbody, 0)
      pltpu.async_copy(rows_cur, acc.at[dstv.at[j]], sem_s_cur, add=True)

    def tbody(t, carry):
      chunk(t * 2, rowsa, ava, bva, exba, sema, semta, semsa, semda,
            rowsb, avb, bvb, exbb, semb, semtb, semsb, semdb)
      chunk(t * 2 + 1, rowsb, avb, bvb, exbb, semb, semtb, semsb, semdb,
            rowsa, ava, bva, exba, sema, semta, semsa, semda)
      return carry

    lax.fori_loop(0, NJ // 2, tbody, 0)
    pltpu.make_async_copy(rowsb, acc.at[dstv.at[0]], semsb).wait()
    pltpu.make_async_copy(exbb, den_sh.at[dstv.at[0]], semdb).wait()
    plsc.subcore_barrier()

    @pl.when(sid == 0)
    def _():
      pltpu.sync_copy(den_sh, den_out.at[cid])

    pltpu.sync_copy(acc.at[sl], agg_out.at[cid, sl])

  return pl.kernel(
      body,
      out_type=[jax.ShapeDtypeStruct((NC, NP), F32),
                jax.ShapeDtypeStruct((NC, NP, d), F32)],
      mesh=_MESH,
      compiler_params=_SC_PARAMS,
      scratch_types=[
          pltpu.VMEM((NJ, CH), I32), pltpu.VMEM((NJ, CH), I32),
          pltpu.VMEM((CH,), F32), pltpu.VMEM((CH,), F32),
          pltpu.VMEM((CH,), F32), pltpu.VMEM((CH,), F32),
          pltpu.VMEM((16,), F32),
          pltpu.VMEM((CH,), F32), pltpu.VMEM((CH,), F32),
          pltpu.VMEM((CH, d), F32), pltpu.VMEM((CH, d), F32),
          pltpu.VMEM_SHARED((NP,), F32), pltpu.VMEM_SHARED((NP,), F32),
          pltpu.VMEM_SHARED((NP,), F32),
          pltpu.VMEM_SHARED((NP, d), F32),
          pltpu.SemaphoreType.DMA, pltpu.SemaphoreType.DMA,
          pltpu.SemaphoreType.DMA, pltpu.SemaphoreType.DMA,
          pltpu.SemaphoreType.DMA, pltpu.SemaphoreType.DMA,
          pltpu.SemaphoreType.DMA, pltpu.SemaphoreType.DMA,
      ],
  )(src3, dst3, ta1, tb1, mvec, xlt, zrows, znp)


_BS = lambda r, c: pl.BlockSpec((r, c), lambda i: (i, 0))
_BC = lambda r, c: pl.BlockSpec((r, c), lambda i: (0, 0))

_TAB_OUT = [jax.ShapeDtypeStruct((NP, 1), F32),
            jax.ShapeDtypeStruct((NP, 1), F32),
            jax.ShapeDtypeStruct((1, 128), F32)]
_TAB_SPECS = [_BS(BLK, 1), _BS(BLK, 1), _BC(1, 128)]


def _emit_tabs(xl, asv, adv, ta_ref, tb_ref, mv_ref):
  asrc = jnp.sum(xl * asv, axis=1, keepdims=True)
  adst = jnp.sum(xl * adv, axis=1, keepdims=True)
  rowid = (lax.broadcasted_iota(I32, (BLK, 1), 0)
           + pl.program_id(0) * BLK)
  ta = jnp.where(rowid < N, asrc, -1e30)
  ta_ref[...] = ta
  tb_ref[...] = adst

  @pl.when(pl.program_id(0) == 0)
  def _():
    mv_ref[...] = jnp.full((1, 128), -3e38, F32)

  mv_ref[...] = jnp.maximum(mv_ref[...], jnp.max(ta))


def _self_exp(ta, tb, m):
  return jnp.exp(_leaky(ta + tb) - _leaky(m + tb))


def _dense0(x, w, ats, atd):
  def body(x_ref, w_ref, as_ref, ad_ref, xl_ref, ta_ref, tb_ref, mv_ref):
    xl = _dot(x_ref[...], w_ref[...])
    xl_ref[...] = xl
    _emit_tabs(xl, as_ref[...], ad_ref[...], ta_ref, tb_ref, mv_ref)

  return pl.pallas_call(
      body, grid=(GRID,),
      in_specs=[_BS(BLK, 128), _BC(128, 128), _BC(1, 128), _BC(1, 128)],
      out_specs=[_BS(BLK, 128)] + _TAB_SPECS,
      out_shape=[jax.ShapeDtypeStruct((NP, 128), F32)] + _TAB_OUT)(
          x, w, ats.reshape(1, 128), atd.reshape(1, 128))


def _agg_h(d0, d1, p0, p1, xl, ta, tb, m, b, g, be):
  dsf = _self_exp(ta, tb, m)
  inv = 1.0 / (d0 + d1 + dsf + 1e-16)
  agg = (p0 + p1) * inv + xl * (dsf * inv) + b
  hb = agg * (g * (1.0 / jnp.sqrt(1.0 + 1e-5))) + be
  return jnp.maximum(hb, 0.0)


def _dense_mid(d0, d1, p0, p1, xl, ta, tb, mvec, b, gam, bet, wn, atsn, atdn):
  def body(d0_ref, d1_ref, p0_ref, p1_ref, xl_ref, tai_ref, tbi_ref, mvi_ref,
           b_ref, g_ref, be_ref, w_ref, as_ref, ad_ref,
           xl2_ref, ta_ref, tb_ref, mv_ref):
    h = _agg_h(d0_ref[...], d1_ref[...], p0_ref[...], p1_ref[...],
               xl_ref[...], tai_ref[...], tbi_ref[...], mvi_ref[0, 0],
               b_ref[...], g_ref[...], be_ref[...])
    xl2 = _dot(h, w_ref[...])
    xl2_ref[...] = xl2
    _emit_tabs(xl2, as_ref[...], ad_ref[...], ta_ref, tb_ref, mv_ref)

  return pl.pallas_call(
      body, grid=(GRID,),
      in_specs=[_BS(BLK, 1), _BS(BLK, 1), _BS(BLK, 128), _BS(BLK, 128),
                _BS(BLK, 128), _BS(BLK, 1), _BS(BLK, 1), _BC(1, 128),
                _BC(1, 128), _BC(1, 128), _BC(1, 128), _BC(128, 128),
                _BC(1, 128), _BC(1, 128)],
      out_specs=[_BS(BLK, 128)] + _TAB_SPECS,
      out_shape=[jax.ShapeDtypeStruct((NP, 128), F32)] + _TAB_OUT)(
          d0, d1, p0, p1, xl, ta, tb, mvec, b.reshape(1, 128),
          gam.reshape(1, 128), bet.reshape(1, 128), wn,
          atsn.reshape(1, 128), atdn.reshape(1, 128))


def _dense_last(d0, d1, p0, p1, xl, ta, tb, mvec, b, gam, bet, w4, w1, b1,
                w2, b2, ats4, atd4):
  def body(d0_ref, d1_ref, p0_ref, p1_ref, xl_ref, tai_ref, tbi_ref, mvi_ref,
           b_ref, g_ref, be_ref, w4_ref, w1_ref, b1_ref, w2_ref, b2_ref,
           as_ref, ad_ref, t_ref, ta_ref, tb_ref, mv_ref):
    h = _agg_h(d0_ref[...], d1_ref[...], p0_ref[...], p1_ref[...],
               xl_ref[...], tai_ref[...], tbi_ref[...], mvi_ref[0, 0],
               b_ref[...], g_ref[...], be_ref[...])
    xl4 = _dot(h, w4_ref[...])
    t = _dot(h, w1_ref[...]) + b1_ref[...]
    t2 = _dot(t, w2_ref[...]) + b2_ref[...]
    t2 = t2 - jnp.max(t2, axis=1, keepdims=True)
    et = jnp.exp(t2)
    sm = et / jnp.sum(et, axis=1, keepdims=True)
    rowid = (lax.broadcasted_iota(I32, (BLK, 1), 0)
             + pl.program_id(0) * BLK)
    s = jnp.where(rowid < N, sm, 0.0)
    tv = jnp.concatenate([xl4, s, jnp.zeros((BLK, 64), F32)], axis=1)
    t_ref[...] = tv
    _emit_tabs(tv, as_ref[...], ad_ref[...], ta_ref, tb_ref, mv_ref)

  return pl.pallas_call(
      body, grid=(GRID,),
      in_specs=[_BS(BLK, 1), _BS(BLK, 1), _BS(BLK, 128), _BS(BLK, 128),
                _BS(BLK, 128), _BS(BLK, 1), _BS(BLK, 1), _BC(1, 128),
                _BC(1, 128), _BC(1, 128), _BC(1, 128), _BC(128, 32),
                _BC(128, 128), _BC(1, 128), _BC(128, 32), _BC(1, 32),
                _BC(1, 128), _BC(1, 128)],
      out_specs=[_BS(BLK, 128)] + _TAB_SPECS,
      out_shape=[jax.ShapeDtypeStruct((NP, 128), F32)] + _TAB_OUT)(
          d0, d1, p0, p1, xl, ta, tb, mvec, b.reshape(1, 128),
          gam.reshape(1, 128), bet.reshape(1, 128), w4, w1,
          b1.reshape(1, 128), w2, b2.reshape(1, 32),
          ats4.reshape(1, 128), atd4.reshape(1, 128))


def _finalize(t, d0, d1, p0, p1, ta, tb, mvec, b, eye):
  c = 32

  def body(t_ref, d0_ref, d1_ref, p0_ref, p1_ref, ta_ref, tb_ref, mv_ref,
           b_ref, eye_ref, z_ref, l_ref):
    tv = t_ref[...]
    xl4 = tv[:, 0:32]
    sv = tv[:, 32:64]
    pv = p0_ref[...] + p1_ref[...]
    dsv = _self_exp(ta_ref[...], tb_ref[...], mv_ref[0, 0])
    inv = 1.0 / (d0_ref[...] + d1_ref[...] + dsv + 1e-16)
    zr = pv[:, 0:32] * inv + xl4 * (dsv * inv) + b_ref[...]
    rn = jnp.sqrt(jnp.sum(zr * zr, axis=1, keepdims=True))
    z_ref[...] = zr / jnp.maximum(rn, 1e-12)

    g = pv[:, 32:64]
    t1 = jnp.sum(g * sv)
    v = jnp.sum(g, axis=0, keepdims=True)
    vv = jnp.sum(v * v)
    m = E / 2.0
    spectral = -(t1 - vv / (2.0 * m)) / (2.0 * m)

    ss = lax.dot_general(sv, sv, (((0,), (0,)), ((), ())),
                         precision=_HIGH, preferred_element_type=F32)
    ssn = jnp.sqrt(jnp.sum(ss * ss))
    dif = ss / ssn - eye_ref[...] / jnp.sqrt(1.0 * c)
    ortho = jnp.sqrt(jnp.sum(dif * dif))

    csz = jnp.sum(sv, axis=0, keepdims=True)
    cluster = jnp.sqrt(jnp.sum(csz * csz)) / N * jnp.sqrt(1.0 * c) - 1.0

    l_ref[...] = jnp.broadcast_to(spectral + ortho + cluster, (1, 1))

  return pl.pallas_call(
      body,
      out_shape=[jax.ShapeDtypeStruct((NP, c), F32),
                 jax.ShapeDtypeStruct((1, 1), F32)])(
                     t, d0, d1, p0, p1, ta, tb, mvec, b.reshape(1, c), eye)


def kernel(x, edge_weight, params, edge_index):
  del edge_weight
  src = edge_index[0].astype(I32)
  dst = edge_index[1].astype(I32)
  pad = N + (jnp.arange(EP - E, dtype=I32) % (NP - N))
  src3 = jnp.concatenate([src, pad]).reshape(NW, NJ, CH)
  dst3 = jnp.concatenate([dst, pad]).reshape(NW, NJ, CH)

  xp = jnp.pad(x, ((0, NP - N), (0, 0)))
  znp = jnp.zeros((NP,), F32)
  zr128 = jnp.zeros((64, 128), F32)
  eye = jnp.eye(32, dtype=F32)
  mp = params['pool']
  p4 = params['conv4']

  xl, ta, tb, mvec = _dense0(xp, params['conv0']['W'],
                             params['conv0']['att_src'],
                             params['conv0']['att_dst'])
  for i in range(4):
    p = params['conv%d' % i]
    denp, outp = _sc_layer(src3, dst3, ta.reshape(NP), tb.reshape(NP),
                           mvec.reshape(128), xl, zr128, znp, 128, 8)
    bn = params['bn%d' % i]
    d0 = denp[0].reshape(NP, 1)
    d1 = denp[1].reshape(NP, 1)
    if i < 3:
      pn = params['conv%d' % (i + 1)]
      xl, ta, tb, mvec = _dense_mid(
          d0, d1, outp[0], outp[1], xl, ta, tb, mvec, p['b'], bn['gamma'],
          bn['beta'], pn['W'], pn['att_src'], pn['att_dst'])
    else:
      t, ta, tb, mvec = _dense_last(
          d0, d1, outp[0], outp[1], xl, ta, tb, mvec, p['b'], bn['gamma'],
          bn['beta'], p4['W'], mp['W1'], mp['b1'], mp['W2'], mp['b2'],
          jnp.pad(p4['att_src'], (0, 96)), jnp.pad(p4['att_dst'], (0, 96)))

  denp4, outp4 = _sc_layer(src3, dst3, ta.reshape(NP), tb.reshape(NP),
                           mvec.reshape(128), t, zr128, znp, 128, 2)

  z_full, loss = _finalize(t, denp4[0].reshape(NP, 1),
                           denp4[1].reshape(NP, 1), outp4[0], outp4[1],
                           ta, tb, mvec, p4['b'], eye)
  s_full = t[:, 32:64]
  return (s_full[:N][None], z_full[:N], loss[0, 0])
